# baseline (speedup 1.0000x reference)
import functools

import jax
import jax.numpy as jnp
from jax import lax
from jax.experimental import pallas as pl
from jax.experimental.pallas import tpu as pltpu
from jax.experimental.pallas import tpu_sc as plsc

_VOCAB = 100000
_EMBED = 128
_BATCH = 1024
_CTX = 50

_NC = 2
_NS = 16
_LANES = 16
_NW = _NC * _NS
_B_PER_W = _BATCH // _NW
_EV = _EMBED // _LANES


def _sc_pool_body(emb_hbm, idx_hbm, out_hbm, idx_v, rows_v, pool_v, sem):
    wid = lax.axis_index("s") * _NC + lax.axis_index("c")
    base = wid * _B_PER_W
    pltpu.sync_copy(idx_hbm.at[pl.ds(base, _B_PER_W)], idx_v)

    def do_row(b, carry):
        pltpu.async_copy(emb_hbm.at[idx_v.at[b]], rows_v, sem).wait()
        scale = 1.0 / _CTX
        for j in range(_EV):
            acc = rows_v[0, pl.ds(j * _LANES, _LANES)]
            for c in range(1, _CTX):
                acc = acc + rows_v[c, pl.ds(j * _LANES, _LANES)]
            pool_v[b, pl.ds(j * _LANES, _LANES)] = acc * scale
        return carry

    lax.fori_loop(0, _B_PER_W, do_row, 0)
    pltpu.sync_copy(pool_v, out_hbm.at[pl.ds(base, _B_PER_W)])


@jax.jit
def _sc_pool(emb_table, idx):
    mesh = plsc.VectorSubcoreMesh(core_axis_name="c", subcore_axis_name="s")
    return pl.kernel(
        _sc_pool_body,
        out_type=jax.ShapeDtypeStruct((_BATCH, _EMBED), jnp.float32),
        mesh=mesh,
        scratch_types=[
            pltpu.VMEM((_B_PER_W, _CTX), jnp.int32),
            pltpu.VMEM((_CTX, _EMBED), jnp.float32),
            pltpu.VMEM((_B_PER_W, _EMBED), jnp.float32),
            pltpu.SemaphoreType.DMA,
        ],
    )(emb_table, idx)


def _matmul_body(x_ref, w_ref, b_ref, o_ref):
    o_ref[...] = (
        jnp.dot(x_ref[...], w_ref[...], preferred_element_type=jnp.float32)
        + b_ref[...]
    )


@jax.jit
def _project(pooled, dense_w, dense_b):
    vt = 2048
    grid = (pl.cdiv(_VOCAB, vt),)
    return pl.pallas_call(
        _matmul_body,
        grid=grid,
        in_specs=[
            pl.BlockSpec((_BATCH, _EMBED), lambda i: (0, 0)),
            pl.BlockSpec((_EMBED, vt), lambda i: (0, i)),
            pl.BlockSpec((1, vt), lambda i: (0, i)),
        ],
        out_specs=pl.BlockSpec((_BATCH, vt), lambda i: (0, i)),
        out_shape=jax.ShapeDtypeStruct((_BATCH, _VOCAB), jnp.float32),
    )(pooled, dense_w, dense_b.reshape(1, _VOCAB))


def kernel(inputs, emb_table, dense_w, dense_b):
    idx = inputs.astype(jnp.int32)
    pooled = _sc_pool(emb_table, idx)
    return _project(pooled, dense_w, dense_b)


# speedup vs baseline: 1.0037x; 1.0037x over previous
import functools

import jax
import jax.numpy as jnp
from jax import lax
from jax.experimental import pallas as pl
from jax.experimental.pallas import tpu as pltpu
from jax.experimental.pallas import tpu_sc as plsc

_VOCAB = 100000
_EMBED = 128
_BATCH = 1024
_CTX = 50

_NC = 2
_NS = 16
_LANES = 16
_NW = _NC * _NS
_B_PER_W = _BATCH // _NW
_EV = _EMBED // _LANES


def _sc_pool_body(emb_hbm, idx_hbm, out_hbm, idx_v, rows_v, pool_v, sem):
    wid = lax.axis_index("s") * _NC + lax.axis_index("c")
    base = wid * _B_PER_W
    pltpu.sync_copy(idx_hbm.at[pl.ds(base, _B_PER_W)], idx_v)

    def do_row(b, carry):
        pltpu.async_copy(emb_hbm.at[idx_v.at[b]], rows_v, sem).wait()
        scale = 1.0 / _CTX
        for j in range(_EV):
            acc = rows_v[0, pl.ds(j * _LANES, _LANES)]
            for c in range(1, _CTX):
                acc = acc + rows_v[c, pl.ds(j * _LANES, _LANES)]
            pool_v[b, pl.ds(j * _LANES, _LANES)] = acc * scale
        return carry

    lax.fori_loop(0, _B_PER_W, do_row, 0)
    pltpu.sync_copy(pool_v, out_hbm.at[pl.ds(base, _B_PER_W)])


@jax.jit
def _sc_pool(emb_table, idx):
    mesh = plsc.VectorSubcoreMesh(core_axis_name="c", subcore_axis_name="s")
    return pl.kernel(
        _sc_pool_body,
        out_type=jax.ShapeDtypeStruct((_BATCH, _EMBED), jnp.float32),
        mesh=mesh,
        scratch_types=[
            pltpu.VMEM((_B_PER_W, _CTX), jnp.int32),
            pltpu.VMEM((_CTX, _EMBED), jnp.float32),
            pltpu.VMEM((_B_PER_W, _EMBED), jnp.float32),
            pltpu.SemaphoreType.DMA,
        ],
    )(emb_table, idx)


def _matmul_body(x_ref, w_ref, b_ref, o_ref):
    o_ref[...] = (
        jnp.dot(x_ref[...], w_ref[...], preferred_element_type=jnp.float32)
        + b_ref[...]
    )


@jax.jit
def _project(pooled, dense_w, dense_b):
    vt = 4096
    grid = (pl.cdiv(_VOCAB, vt),)
    return pl.pallas_call(
        _matmul_body,
        grid=grid,
        in_specs=[
            pl.BlockSpec((_BATCH, _EMBED), lambda i: (0, 0)),
            pl.BlockSpec((_EMBED, vt), lambda i: (0, i)),
            pl.BlockSpec((1, vt), lambda i: (0, i)),
        ],
        out_specs=pl.BlockSpec((_BATCH, vt), lambda i: (0, i)),
        out_shape=jax.ShapeDtypeStruct((_BATCH, _VOCAB), jnp.float32),
    )(pooled, dense_w, dense_b.reshape(1, _VOCAB))


def kernel(inputs, emb_table, dense_w, dense_b):
    idx = inputs.astype(jnp.int32)
    pooled = _sc_pool(emb_table, idx)
    return _project(pooled, dense_w, dense_b)
